# Initial kernel scaffold; baseline (speedup 1.0000x reference)
import jax
import jax.numpy as jnp
from jax.experimental import pallas as pl


def kernel(x, ijk, W_mid, W_out):
    raise NotImplementedError("write your pallas kernel here")


# speedup vs baseline: 76.0833x; 76.0833x over previous
import functools

import jax
import jax.numpy as jnp
from jax import lax
from jax.experimental import pallas as pl
from jax.experimental.pallas import tpu as pltpu
from jax.experimental.pallas import tpu_sc as plsc

SF = 2
S = SF ** 3
D = 64
N = 16384
IN_CH = 512
MID_CH = 512
C = MID_CH // S
OUT_CH = 64
NS = N * S

M_TILE = 512


def _mm_body(x_ref, wmid_ref, wout_ref, y_ref, wcomb_ref):
    @pl.when(pl.program_id(0) == 0)
    def _():
        wout = wout_ref[...]
        for l in range(S):
            wcomb_ref[:, l * C:(l + 1) * C] = jnp.dot(
                wmid_ref[:, l * C:(l + 1) * C], wout,
                preferred_element_type=jnp.float32)

    y_ref[...] = jnp.dot(x_ref[...], wcomb_ref[...],
                         preferred_element_type=jnp.float32)


_matmul = pl.pallas_call(
    _mm_body,
    grid=(N // M_TILE,),
    in_specs=[
        pl.BlockSpec((M_TILE, IN_CH), lambda m: (m, 0)),
        pl.BlockSpec((IN_CH, MID_CH), lambda m: (0, 0)),
        pl.BlockSpec((C, OUT_CH), lambda m: (0, 0)),
    ],
    out_specs=pl.BlockSpec((M_TILE, MID_CH), lambda m: (m, 0)),
    out_shape=jax.ShapeDtypeStruct((N, MID_CH), jnp.float32),
    scratch_shapes=[pltpu.VMEM((IN_CH, MID_CH), jnp.float32)],
)

_NC = 2
_NSUB = 16
_L = 16
_NW = _NC * _NSUB
P_PER_W = N // _NW
K_PER_SUB = N // _NSUB
CHUNK = 16
NCHUNK = P_PER_W // CHUNK


def _searchsorted(keys_ref, q):
    def step(_, lohi):
        lo, hi = lohi
        mid = (lo + hi) >> 1
        kv = plsc.load_gather(keys_ref, [mid])
        pred = kv < q
        return (jnp.where(pred, mid + 1, lo), jnp.where(pred, hi, mid))

    lo0 = jnp.zeros((_L,), jnp.int32)
    hi0 = jnp.full((_L,), N, jnp.int32)
    lo, _ = lax.fori_loop(0, 14, step, (lo0, hi0))
    return lo


_sc_mesh = plsc.VectorSubcoreMesh(core_axis_name="c", subcore_axis_name="s")


_PAIRS = CHUNK * S // 2


@functools.partial(
    pl.kernel,
    out_type=jax.ShapeDtypeStruct((NS // 2, 2 * OUT_CH), jnp.float32),
    mesh=_sc_mesh,
    scratch_types=[
        pltpu.VMEM((K_PER_SUB * 3,), jnp.int32),
        pltpu.VMEM((K_PER_SUB,), jnp.int32),
        pltpu.VMEM((N,), jnp.int32),
        pltpu.VMEM_SHARED((N,), jnp.int32),
        pltpu.VMEM((_PAIRS, 2 * OUT_CH), jnp.float32),
        pltpu.VMEM((_PAIRS,), jnp.int32),
        pltpu.SemaphoreType.DMA,
    ],
    compiler_params=pltpu.CompilerParams(needs_layout_passes=False),
)
def _sc_scatter(ijk_hbm, y_hbm, out_hbm, own_ijk, mykeys, keys, keys_sh,
                rows, idxb, sem):
    c = lax.axis_index("c")
    s = lax.axis_index("s")
    lanes = lax.iota(jnp.int32, _L)

    pltpu.sync_copy(ijk_hbm.at[pl.ds(s * K_PER_SUB * 3, K_PER_SUB * 3)],
                    own_ijk)

    def key_step(v, _):
        f16 = (v * _L + lanes) * 3
        i = plsc.load_gather(own_ijk, [f16])
        j = plsc.load_gather(own_ijk, [f16 + 1])
        k = plsc.load_gather(own_ijk, [f16 + 2])
        mykeys[pl.ds(v * _L, _L)] = i * (D * D) + j * D + k
        return 0

    lax.fori_loop(0, K_PER_SUB // _L, key_step, 0)
    pltpu.sync_copy(mykeys, keys_sh.at[pl.ds(s * K_PER_SUB, K_PER_SUB)])
    plsc.subcore_barrier()
    pltpu.sync_copy(keys_sh, keys)

    w0 = s * K_PER_SUB + c * P_PER_W

    def chunk_body(t, _):
        f16 = (c * P_PER_W + t * CHUNK + lanes) * 3
        i = plsc.load_gather(own_ijk, [f16])
        j = plsc.load_gather(own_ijk, [f16 + 1])
        p = w0 + t * CHUNK + lanes
        iv = i * (D * D)
        s1 = _searchsorted(keys, iv)
        s2 = _searchsorted(keys, iv + j * D)
        s3 = _searchsorted(keys, iv + D * D)
        s4 = _searchsorted(keys, iv + j * D + D)
        base = 4 * s1 + 2 * (s2 - s1) + (p - s2)
        c_di = 2 * (s3 - s1)
        c_dj = s4 - s2
        for l2 in range(S // 2):
            r = base
            if l2 & 2:
                r = r + c_di
            if l2 & 1:
                r = r + c_dj
            plsc.store_scatter(idxb, [lanes * (S // 2) + l2], r)
        pltpu.sync_copy(
            y_hbm.at[pl.ds((w0 + t * CHUNK) * (S // 2), _PAIRS)], rows)
        pltpu.async_copy(rows, out_hbm.at[idxb], sem).wait()
        return 0

    lax.fori_loop(0, NCHUNK, chunk_body, 0)


def kernel(x, ijk, W_mid, W_out):
    y = _matmul(x, W_mid, W_out)
    y2 = y.reshape(NS // 2, 2 * OUT_CH)
    ijk_flat = ijk.astype(jnp.int32).reshape(N * 3)
    out = _sc_scatter(ijk_flat, y2)
    return out.reshape(NS, OUT_CH)


# speedup vs baseline: 95.4925x; 1.2551x over previous
import functools

import jax
import jax.numpy as jnp
from jax import lax
from jax.experimental import pallas as pl
from jax.experimental.pallas import tpu as pltpu
from jax.experimental.pallas import tpu_sc as plsc

SF = 2
S = SF ** 3
D = 64
N = 16384
IN_CH = 512
MID_CH = 512
C = MID_CH // S
OUT_CH = 64
NS = N * S
NQ = S // 2
PW = 2 * OUT_CH

M_TILE = 512


def _mm_body(x_ref, wmid_ref, wout_ref, y_ref, wcomb_ref):
    @pl.when(pl.program_id(0) == 0)
    def _():
        wout = wout_ref[...]
        for l in range(S):
            wcomb_ref[:, l * C:(l + 1) * C] = jnp.dot(
                wmid_ref[:, l * C:(l + 1) * C], wout,
                preferred_element_type=jnp.float32)

    x = x_ref[...]
    for q in range(NQ):
        y_ref[q, :, :] = jnp.dot(x, wcomb_ref[:, q * PW:(q + 1) * PW],
                                 preferred_element_type=jnp.float32)


_matmul = pl.pallas_call(
    _mm_body,
    grid=(N // M_TILE,),
    in_specs=[
        pl.BlockSpec((M_TILE, IN_CH), lambda m: (m, 0)),
        pl.BlockSpec((IN_CH, MID_CH), lambda m: (0, 0)),
        pl.BlockSpec((C, OUT_CH), lambda m: (0, 0)),
    ],
    out_specs=pl.BlockSpec((NQ, M_TILE, PW), lambda m: (0, m, 0)),
    out_shape=jax.ShapeDtypeStruct((NQ, N, PW), jnp.float32),
    scratch_shapes=[pltpu.VMEM((IN_CH, MID_CH), jnp.float32)],
)

_NC = 2
_NSUB = 16
_L = 16
_NW = _NC * _NSUB
P_PER_W = N // _NW
K_PER_SUB = N // _NSUB
CHUNK = 32
_PAIRS = CHUNK * NQ
NCHUNK = P_PER_W // CHUNK
_VPC = CHUNK // _L


def _searchsorted(keys_ref, q):
    def step(_, lohi):
        lo, hi = lohi
        mid = (lo + hi) >> 1
        kv = plsc.load_gather(keys_ref, [mid])
        pred = kv < q
        return (jnp.where(pred, mid + 1, lo), jnp.where(pred, hi, mid))

    lo0 = jnp.zeros((_L,), jnp.int32)
    hi0 = jnp.full((_L,), N, jnp.int32)
    lo, _ = lax.fori_loop(0, 14, step, (lo0, hi0))
    return lo


_sc_mesh = plsc.VectorSubcoreMesh(core_axis_name="c", subcore_axis_name="s")


@functools.partial(
    pl.kernel,
    out_type=jax.ShapeDtypeStruct((NS // 2, PW), jnp.float32),
    mesh=_sc_mesh,
    scratch_types=[
        pltpu.VMEM((K_PER_SUB * 3,), jnp.int32),
        pltpu.VMEM((K_PER_SUB,), jnp.int32),
        pltpu.VMEM((N,), jnp.int32),
        pltpu.VMEM_SHARED((N,), jnp.int32),
        pltpu.VMEM((_PAIRS, PW), jnp.float32),
        pltpu.VMEM((_PAIRS, PW), jnp.float32),
        pltpu.VMEM((_PAIRS,), jnp.int32),
        pltpu.VMEM((_PAIRS,), jnp.int32),
        pltpu.SemaphoreType.DMA,
        pltpu.SemaphoreType.DMA,
        pltpu.SemaphoreType.DMA,
        pltpu.SemaphoreType.DMA,
    ],
    compiler_params=pltpu.CompilerParams(needs_layout_passes=False),
)
def _sc_scatter(ijk_hbm, y_hbm, out_hbm, own_ijk, mykeys, keys, keys_sh,
                rows0, rows1, idx0, idx1, semi0, semi1, semo0, semo1):
    c = lax.axis_index("c")
    s = lax.axis_index("s")
    lanes = lax.iota(jnp.int32, _L)

    pltpu.sync_copy(ijk_hbm.at[pl.ds(s * K_PER_SUB * 3, K_PER_SUB * 3)],
                    own_ijk)

    def key_step(v, _):
        f16 = (v * _L + lanes) * 3
        i = plsc.load_gather(own_ijk, [f16])
        j = plsc.load_gather(own_ijk, [f16 + 1])
        k = plsc.load_gather(own_ijk, [f16 + 2])
        mykeys[pl.ds(v * _L, _L)] = i * (D * D) + j * D + k
        return 0

    lax.fori_loop(0, K_PER_SUB // _L, key_step, 0)
    pltpu.sync_copy(mykeys, keys_sh.at[pl.ds(s * K_PER_SUB, K_PER_SUB)])
    plsc.subcore_barrier()
    pltpu.sync_copy(keys_sh, keys)

    w0 = s * K_PER_SUB + c * P_PER_W
    bufs = ((rows0, idx0, semi0, semo0), (rows1, idx1, semi1, semo1))

    def start_in(t, rows, semi):
        p0 = w0 + t * CHUNK
        return [pltpu.async_copy(y_hbm.at[pl.ds(q * N + p0, CHUNK)],
                                 rows.at[pl.ds(q * CHUNK, CHUNK)], semi)
                for q in range(NQ)]

    def ranks(t, idxb):
        for v in range(_VPC):
            f16 = (c * P_PER_W + t * CHUNK + v * _L + lanes) * 3
            i = plsc.load_gather(own_ijk, [f16])
            j = plsc.load_gather(own_ijk, [f16 + 1])
            p = w0 + t * CHUNK + v * _L + lanes
            iv = i * (D * D)
            s1 = _searchsorted(keys, iv)
            s2 = _searchsorted(keys, iv + j * D)
            s3 = _searchsorted(keys, iv + D * D)
            s4 = _searchsorted(keys, iv + j * D + D)
            base = 4 * s1 + 2 * (s2 - s1) + (p - s2)
            c_di = 2 * (s3 - s1)
            c_dj = s4 - s2
            for q in range(NQ):
                r = base
                if q & 2:
                    r = r + c_di
                if q & 1:
                    r = r + c_dj
                plsc.store_scatter(idxb, [q * CHUNK + v * _L + lanes], r)

    def do_chunk(t, buf, first):
        rows, idxb, semi, semo = buf
        if not first:
            pltpu.make_async_copy(rows, out_hbm.at[idxb], semo).wait()
        descs = start_in(t, rows, semi)
        ranks(t, idxb)
        for d in descs:
            d.wait()
        pltpu.async_copy(rows, out_hbm.at[idxb], semo)

    do_chunk(0, bufs[0], True)
    do_chunk(1, bufs[1], True)

    def body(u, _):
        do_chunk(2 * u, bufs[0], False)
        do_chunk(2 * u + 1, bufs[1], False)
        return 0

    lax.fori_loop(1, NCHUNK // 2, body, 0)
    pltpu.make_async_copy(rows0, out_hbm.at[idx0], semo0).wait()
    pltpu.make_async_copy(rows1, out_hbm.at[idx1], semo1).wait()


def kernel(x, ijk, W_mid, W_out):
    y = _matmul(x, W_mid, W_out)
    y2 = y.reshape(NQ * N, PW)
    ijk_flat = ijk.astype(jnp.int32).reshape(N * 3)
    out = _sc_scatter(ijk_flat, y2)
    return out.reshape(NS, OUT_CH)


# speedup vs baseline: 101.9802x; 1.0679x over previous
import functools

import jax
import jax.numpy as jnp
from jax import lax
from jax.experimental import pallas as pl
from jax.experimental.pallas import tpu as pltpu
from jax.experimental.pallas import tpu_sc as plsc

SF = 2
S = SF ** 3
D = 64
N = 16384
IN_CH = 512
MID_CH = 512
C = MID_CH // S
OUT_CH = 64
NS = N * S
NQ = S // 2
PW = 2 * OUT_CH

M_TILE = 512


def _mm_body(x_ref, wmid_ref, wout_ref, y_ref, wcomb_ref):
    @pl.when(pl.program_id(0) == 0)
    def _():
        wout = wout_ref[...]
        for l in range(S):
            wcomb_ref[:, l * C:(l + 1) * C] = jnp.dot(
                wmid_ref[:, l * C:(l + 1) * C], wout,
                preferred_element_type=jnp.float32)

    x = x_ref[...]
    for q in range(NQ):
        y_ref[q, :, :] = jnp.dot(x, wcomb_ref[:, q * PW:(q + 1) * PW],
                                 preferred_element_type=jnp.float32)


_matmul = pl.pallas_call(
    _mm_body,
    grid=(N // M_TILE,),
    in_specs=[
        pl.BlockSpec((M_TILE, IN_CH), lambda m: (m, 0)),
        pl.BlockSpec((IN_CH, MID_CH), lambda m: (0, 0)),
        pl.BlockSpec((C, OUT_CH), lambda m: (0, 0)),
    ],
    out_specs=pl.BlockSpec((NQ, M_TILE, PW), lambda m: (0, m, 0)),
    out_shape=jax.ShapeDtypeStruct((NQ, N, PW), jnp.float32),
    scratch_shapes=[pltpu.VMEM((IN_CH, MID_CH), jnp.float32)],
)

_NC = 2
_NSUB = 16
_L = 16
_NW = _NC * _NSUB
P_PER_W = N // _NW
CHUNK = 32
_PAIRS = CHUNK * NQ
NCHUNK = P_PER_W // CHUNK


def _searchsorted(keys_ref, q):
    def step(_, lohi):
        lo, hi = lohi
        mid = (lo + hi) >> 1
        kv = plsc.load_gather(keys_ref, [mid])
        pred = kv < q
        return (jnp.where(pred, mid + 1, lo), jnp.where(pred, hi, mid))

    lo0 = jnp.zeros((_L,), jnp.int32)
    hi0 = jnp.full((_L,), N, jnp.int32)
    lo, _ = lax.fori_loop(0, 14, step, (lo0, hi0))
    return lo


_sc_mesh = plsc.VectorSubcoreMesh(core_axis_name="c", subcore_axis_name="s")


@functools.partial(
    pl.kernel,
    out_type=jax.ShapeDtypeStruct((NQ * N,), jnp.int32),
    mesh=_sc_mesh,
    scratch_types=[
        pltpu.VMEM((N,), jnp.int32),
        pltpu.VMEM((NQ * P_PER_W,), jnp.int32),
    ],
    compiler_params=pltpu.CompilerParams(needs_layout_passes=False),
)
def _sc_ranks(key_hbm, fwd_hbm, keys, fwdbuf):
    c = lax.axis_index("c")
    s = lax.axis_index("s")
    lanes = lax.iota(jnp.int32, _L)
    pltpu.sync_copy(key_hbm, keys)
    w0 = (s * _NC + c) * P_PER_W

    def vreg_body(v, _):
        kp = keys[pl.ds(w0 + v * _L, _L)]
        i = kp >> (2 * 6)
        j = (kp >> 6) & (D - 1)
        p = w0 + v * _L + lanes
        iv = i * (D * D)
        s1 = _searchsorted(keys, iv)
        s2 = _searchsorted(keys, iv + j * D)
        s3 = _searchsorted(keys, iv + D * D)
        s4 = _searchsorted(keys, iv + j * D + D)
        base = 4 * s1 + 2 * (s2 - s1) + (p - s2)
        c_di = 2 * (s3 - s1)
        c_dj = s4 - s2
        for q in range(NQ):
            r = base
            if q & 2:
                r = r + c_di
            if q & 1:
                r = r + c_dj
            fwdbuf[pl.ds(q * P_PER_W + v * _L, _L)] = r
        return 0

    lax.fori_loop(0, P_PER_W // _L, vreg_body, 0)
    for q in range(NQ):
        pltpu.sync_copy(fwdbuf.at[pl.ds(q * P_PER_W, P_PER_W)],
                        fwd_hbm.at[pl.ds(q * N + w0, P_PER_W)])


@functools.partial(
    pl.kernel,
    out_type=jax.ShapeDtypeStruct((NS // 2, PW), jnp.float32),
    mesh=_sc_mesh,
    scratch_types=[
        pltpu.VMEM((_PAIRS, PW), jnp.float32),
        pltpu.VMEM((_PAIRS, PW), jnp.float32),
        pltpu.VMEM((_PAIRS,), jnp.int32),
        pltpu.VMEM((_PAIRS,), jnp.int32),
        pltpu.SemaphoreType.DMA,
        pltpu.SemaphoreType.DMA,
        pltpu.SemaphoreType.DMA,
        pltpu.SemaphoreType.DMA,
    ],
    compiler_params=pltpu.CompilerParams(needs_layout_passes=False),
)
def _sc_scatter(fwd_hbm, y_hbm, out_hbm, rows0, rows1, idx0, idx1,
                semi0, semi1, semo0, semo1):
    c = lax.axis_index("c")
    s = lax.axis_index("s")
    w0 = (s * _NC + c) * P_PER_W
    bufs = ((rows0, idx0, semi0, semo0), (rows1, idx1, semi1, semo1))

    def start_in(t, rows, idxb, semi):
        p0 = w0 + t * CHUNK
        ds = []
        for q in range(NQ):
            ds.append(pltpu.async_copy(
                y_hbm.at[pl.ds(q * N + p0, CHUNK)],
                rows.at[pl.ds(q * CHUNK, CHUNK)], semi))
            ds.append(pltpu.async_copy(
                fwd_hbm.at[pl.ds(q * N + p0, CHUNK)],
                idxb.at[pl.ds(q * CHUNK, CHUNK)], semi))
        return ds

    def do_chunk(t, buf, first):
        rows, idxb, semi, semo = buf
        if not first:
            pltpu.make_async_copy(rows, out_hbm.at[idxb], semo).wait()
        descs = start_in(t, rows, idxb, semi)
        for d in descs:
            d.wait()
        pltpu.async_copy(rows, out_hbm.at[idxb], semo)

    do_chunk(0, bufs[0], True)
    do_chunk(1, bufs[1], True)

    def body(u, _):
        do_chunk(2 * u, bufs[0], False)
        do_chunk(2 * u + 1, bufs[1], False)
        return 0

    lax.fori_loop(1, NCHUNK // 2, body, 0)
    pltpu.make_async_copy(rows0, out_hbm.at[idx0], semo0).wait()
    pltpu.make_async_copy(rows1, out_hbm.at[idx1], semo1).wait()


def kernel(x, ijk, W_mid, W_out):
    ijk32 = ijk.astype(jnp.int32)
    key = ijk32[:, 0] * (D * D) + ijk32[:, 1] * D + ijk32[:, 2]
    fwd = _sc_ranks(key)
    y = _matmul(x, W_mid, W_out)
    y2 = y.reshape(NQ * N, PW)
    out = _sc_scatter(fwd, y2)
    return out.reshape(NS, OUT_CH)
